# baseline (speedup 1.0000x reference)
import functools

import jax
import jax.numpy as jnp
from jax import lax
from jax.experimental import pallas as pl
from jax.experimental.pallas import tpu as pltpu
from jax.experimental.pallas import tpu_sc as plsc

_NC = 2
_NS = 16
_CHUNK = 80
_F = 128


def _leaky(v):
    return jnp.where(v >= 0, v, 0.01 * v)


def _dot(a, b):
    return jnp.dot(a, b, preferred_element_type=jnp.float32,
                   precision=lax.Precision.HIGHEST)


@functools.lru_cache(maxsize=None)
def _make_degree(nchunks, n):
    cpt = nchunks // (_NC * _NS)
    nblk = 5
    bchunk = cpt // nblk
    mesh = plsc.VectorSubcoreMesh(core_axis_name="c", subcore_axis_name="s")

    @functools.partial(
        pl.kernel,
        out_type=jax.ShapeDtypeStruct((_NC * n,), jnp.float32),
        mesh=mesh,
        scratch_types=[
            pltpu.VMEM((bchunk, _CHUNK), jnp.int32),
            pltpu.VMEM((_CHUNK,), jnp.float32),
            pltpu.VMEM((2000,), jnp.float32),
            pltpu.VMEM_SHARED((n,), jnp.float32),
        ],
    )
    def deg_kernel(col_hbm, out_hbm, col_v, ones_v, zb, acc):
        cid = lax.axis_index("c")
        sid = lax.axis_index("s")
        tid = cid * _NS + sid

        one = jnp.full((16,), 1.0, jnp.float32)
        for j in range(_CHUNK // 16):
            ones_v[pl.ds(j * 16, 16)] = one
        zero = jnp.zeros((16,), jnp.float32)

        def zb_body(i, carry):
            zb[pl.ds(i * 16, 16)] = zero
            return carry

        lax.fori_loop(0, 2000 // 16, zb_body, 0)

        @pl.when(sid == 0)
        def _():
            for q in range(n // 2000):
                pltpu.sync_copy(zb, acc.at[pl.ds(q * 2000, 2000)])

        plsc.subcore_barrier()

        def blk_body(b, carry):
            pltpu.sync_copy(col_hbm.at[tid, b], col_v)

            def body(k, c2):
                pltpu.sync_copy(ones_v, acc.at[col_v.at[k]], add=True)
                return c2

            lax.fori_loop(0, bchunk, body, 0)
            return carry

        lax.fori_loop(0, nblk, blk_body, 0)
        plsc.subcore_barrier()

        @pl.when(sid == 0)
        def _():
            for q in range(n // 2000):
                pltpu.sync_copy(acc.at[pl.ds(q * 2000, 2000)], zb)
                pltpu.sync_copy(zb, out_hbm.at[pl.ds(cid * n + q * 2000, 2000)])

    return deg_kernel


@functools.lru_cache(maxsize=None)
def _make_scatter(nchunks, n):
    cpt = nchunks // (_NC * _NS)
    slabs = n // _CHUNK
    spt_lo = slabs // _NS
    extra = slabs % _NS
    mesh = plsc.VectorSubcoreMesh(core_axis_name="c", subcore_axis_name="s")

    @functools.partial(
        pl.kernel,
        out_type=jax.ShapeDtypeStruct((_NC, n, _F), jnp.float32),
        mesh=mesh,
        compiler_params=pltpu.CompilerParams(use_tc_tiling_on_sc=False),
        scratch_types=(
            [pltpu.VMEM((cpt, _CHUNK), jnp.int32)]
            + [pltpu.VMEM((_CHUNK,), jnp.int32)] * 4
            + [pltpu.VMEM((_CHUNK, _F // 2), jnp.int32)] * 2
            + [pltpu.VMEM((_CHUNK, _F), jnp.float32)] * 2
            + [pltpu.VMEM_SHARED((n, _F), jnp.float32)]
            + [pltpu.SemaphoreType.DMA] * 4
        ),
    )
    def scat_kernel(mb_hbm, rc_hbm, out_hbm, rc_v,
                    rb0, rb1, cb0, cb1, gb0, gb1, sb0, sb1, acc,
                    sg0, sg1, ss0, ss1):
        rbs, cbs = (rb0, rb1), (cb0, cb1)
        gbs, sbs = (gb0, gb1), (sb0, sb1)
        sgs, sss = (sg0, sg1), (ss0, ss1)
        cid = lax.axis_index("c")
        sid = lax.axis_index("s")
        tid = cid * _NS + sid
        nslab = spt_lo + (sid < extra).astype(jnp.int32)

        def unpack_rows(k, rb):
            for j in range(_CHUNK // 16):
                p = rc_v[k, pl.ds(j * 16, 16)]
                rb[pl.ds(j * 16, 16)] = lax.shift_right_logical(p, 16)

        def unpack_cols(k, cb):
            for j in range(_CHUNK // 16):
                p = rc_v[k, pl.ds(j * 16, 16)]
                cb[pl.ds(j * 16, 16)] = lax.bitwise_and(p, 0xFFFF)

        def convert(gb, sb):
            himask = jnp.full((16,), -65536, jnp.int32)

            def conv_body(q, carry):
                r0 = 8 * q
                for rr in range(8):
                    for blk in range(_F // 32):
                        w = gb[r0 + rr, pl.ds(blk * 16, 16)]
                        lo = lax.bitcast_convert_type(
                            lax.shift_left(w, 16), jnp.float32)
                        hi = lax.bitcast_convert_type(
                            lax.bitwise_and(w, himask), jnp.float32)
                        sb[r0 + rr, pl.ds(blk * 32, 16)] = lo
                        sb[r0 + rr, pl.ds(blk * 32 + 16, 16)] = hi
                return carry

            lax.fori_loop(0, _CHUNK // 8, conv_body, 0)

        def gath(b, sem):
            pltpu.async_copy(mb_hbm.at[rbs[b]], gbs[b], sem)

        def gath_wait(b, sem):
            pltpu.make_async_copy(mb_hbm.at[rbs[b]], gbs[b], sem).wait()

        def scat(b, sem):
            pltpu.async_copy(sbs[b], acc.at[cbs[b]], sem, add=True)

        def scat_wait(b, sem):
            pltpu.make_async_copy(sbs[b], acc.at[cbs[b]], sem).wait()

        pltpu.sync_copy(rc_hbm.at[tid], rc_v)
        for b in range(2):
            unpack_rows(b, rbs[b])
            gath(b, sgs[b])

        zero = jnp.zeros((16,), jnp.float32)
        groups = _F // 16

        def zb_body(i, carry):
            sb1[i // groups, pl.ds((i % groups) * 16, 16)] = zero
            return carry

        lax.fori_loop(0, _CHUNK * groups, zb_body, 0)

        def zslab_body(q, carry):
            slab = sid + q * _NS
            pltpu.sync_copy(sb1, acc.at[pl.ds(slab * _CHUNK, _CHUNK)])
            return carry

        lax.fori_loop(0, nslab, zslab_body, 0)
        plsc.subcore_barrier()

        niters = (cpt + 1) // 2

        def body(j, carry):
            for b in range(2):
                k = 2 * j + b

                @pl.when(k < cpt)
                def _(b=b, k=k):
                    gath_wait(b, sgs[b])

                    @pl.when(k >= 2)
                    def _():
                        scat_wait(b, sss[b])

                    convert(gbs[b], sbs[b])
                    unpack_cols(k, cbs[b])
                    scat(b, sss[b])

                    @pl.when(k + 2 < cpt)
                    def _():
                        unpack_rows(k + 2, rbs[b])
                        gath(b, sgs[b])

            return carry

        lax.fori_loop(0, niters, body, 0)
        for b in range(2):
            scat_wait(b, sss[b])
        plsc.subcore_barrier()

        def ex_body(q, carry):
            slab = sid + q * _NS
            pltpu.sync_copy(acc.at[pl.ds(slab * _CHUNK, _CHUNK)],
                            out_hbm.at[cid, pl.ds(slab * _CHUNK, _CHUNK)])
            return carry

        lax.fori_loop(0, nslab, ex_body, 0)

    return scat_kernel


def _tc_pre(degp, x, W1):
    n = x.shape[0]

    def body(degp_ref, x_ref, w_ref, dis_ref, m_ref):
        deg = degp_ref[0] + degp_ref[1] + 1.0
        dis = lax.rsqrt(deg)
        dis_ref[...] = dis
        m_ref[...] = dis * _dot(x_ref[...], w_ref[...])

    return pl.pallas_call(
        body,
        out_shape=(jax.ShapeDtypeStruct((n, 1), jnp.float32),
                   jax.ShapeDtypeStruct((n, _F), jnp.float32)),
    )(degp, x, W1)


def _tc_mid(p, m, dis, b, Wn):
    n = m.shape[0]

    def body(p_ref, m_ref, dis_ref, b_ref, w_ref, out_ref):
        dis = dis_ref[...]
        pre = dis * (p_ref[0] + p_ref[1] + m_ref[...]) + b_ref[...]
        h = _leaky(pre)
        out_ref[...] = dis * _dot(h, w_ref[...])

    return pl.pallas_call(
        body,
        out_shape=jax.ShapeDtypeStruct((n, _F), jnp.float32),
    )(p, m, dis, b, Wn)


def _tc_final(p, m, dis, b3, batch, gfeat, Wg, bg, Wf, bf, Wm1, bm1, Wm2,
              bm2, alpha):
    n = m.shape[0]
    g = gfeat.shape[0]

    def _ln(v):
        mu = jnp.mean(v, axis=-1, keepdims=True)
        var = jnp.mean((v - mu) ** 2, axis=-1, keepdims=True)
        return (v - mu) * lax.rsqrt(var + 1e-5)

    def body(p_ref, m_ref, dis_ref, b_ref, batch_ref, gf_ref, wg_ref, bg_ref,
             wf_ref, bf_ref, wm1_ref, bm1_ref, wm2_ref, bm2_ref, a_ref,
             out_ref):
        dis = dis_ref[...]
        pre = dis * (p_ref[0] + p_ref[1] + m_ref[...]) + b_ref[...]
        h = _leaky(pre)
        gid = lax.broadcasted_iota(jnp.int32, (g, n), 0)
        onehot = (gid == batch_ref[...]).astype(jnp.float32)
        sums = _dot(onehot, h)
        cnt = jnp.sum(onehot, axis=1, keepdims=True)
        pooled = sums / jnp.maximum(cnt, 1.0)
        alpha = 1.0 / (1.0 + jnp.exp(-a_ref[0, 0]))
        gnn = _ln(_dot(pooled, wg_ref[...]) + bg_ref[...])
        gfe = _ln(_dot(gf_ref[...], wf_ref[...]) + bf_ref[...])
        fused = jnp.concatenate([gnn * alpha, gfe * (1.0 - alpha)], axis=1)
        o = _leaky(fused)
        o = _leaky(_dot(o, wm1_ref[...]) + bm1_ref[...])
        out_ref[...] = _dot(o, wm2_ref[...]) + bm2_ref[...]

    out_f = Wm2.shape[1]
    return pl.pallas_call(
        body,
        out_shape=jax.ShapeDtypeStruct((g, out_f), jnp.float32),
    )(p, m, dis, b3, batch, gfeat, Wg, bg, Wf, bf, Wm1, bm1, Wm2, bm2, alpha)


def _to_gather_bf16(m):
    n = m.shape[0]
    mperm = jnp.transpose(m.reshape(n, _F // 32, 2, 16),
                          (0, 1, 3, 2)).reshape(n, _F // 2, 2)
    return lax.bitcast_convert_type(mperm.astype(jnp.bfloat16), jnp.int32)


def kernel(x, edge_index, batch, graph_feature, W1, b1, W2, b2, W3, b3,
           Wg, bg, Wf, bf, Wm1, bm1, Wm2, bm2, alpha_param):
    n = x.shape[0]
    e = edge_index.shape[1]
    nw = _NC * _NS
    cpt = e // (nw * _CHUNK)
    rc3 = (lax.shift_left(edge_index[0], 16) |
           edge_index[1]).reshape(nw, cpt, _CHUNK)
    col4 = edge_index[1].reshape(nw, 5, cpt // 5, _CHUNK)

    degp = _make_degree(e // _CHUNK, n)(col4)
    degp = degp.reshape(_NC, n, 1)
    dis, m1 = _tc_pre(degp, x, W1)

    scat = _make_scatter(e // _CHUNK, n)
    p1 = scat(_to_gather_bf16(m1), rc3)
    m2 = _tc_mid(p1, m1, dis, b1.reshape(1, _F), W2)
    p2 = scat(_to_gather_bf16(m2), rc3)
    m3 = _tc_mid(p2, m2, dis, b2.reshape(1, _F), W3)
    p3 = scat(_to_gather_bf16(m3), rc3)

    return _tc_final(p3, m3, dis, b3.reshape(1, _F), batch.reshape(1, n),
                     graph_feature, Wg, bg.reshape(1, _F), Wf,
                     bf.reshape(1, _F), Wm1, bm1.reshape(1, _F), Wm2,
                     bm2.reshape(1, -1), alpha_param.reshape(1, 1))


# speedup vs baseline: 1.7323x; 1.7323x over previous
import functools

import jax
import jax.numpy as jnp
from jax import lax
from jax.experimental import pallas as pl
from jax.experimental.pallas import tpu as pltpu
from jax.experimental.pallas import tpu_sc as plsc

_NC = 2
_NS = 16
_CHUNK = 80
_F = 128


def _leaky(v):
    return jnp.where(v >= 0, v, 0.01 * v)


def _dot(a, b):
    return jnp.dot(a, b, preferred_element_type=jnp.float32,
                   precision=lax.Precision.HIGHEST)


@functools.lru_cache(maxsize=None)
def _make_degree(nchunks, n):
    cpt = nchunks // (_NC * _NS)
    nblk = 5
    bchunk = cpt // nblk
    mesh = plsc.VectorSubcoreMesh(core_axis_name="c", subcore_axis_name="s")

    @functools.partial(
        pl.kernel,
        out_type=jax.ShapeDtypeStruct((_NC * n,), jnp.float32),
        mesh=mesh,
        scratch_types=[
            pltpu.VMEM((bchunk, _CHUNK), jnp.int32),
            pltpu.VMEM((_CHUNK,), jnp.float32),
            pltpu.VMEM((2000,), jnp.float32),
            pltpu.VMEM_SHARED((n,), jnp.float32),
        ],
    )
    def deg_kernel(col_hbm, out_hbm, col_v, ones_v, zb, acc):
        cid = lax.axis_index("c")
        sid = lax.axis_index("s")
        tid = cid * _NS + sid

        one = jnp.full((16,), 1.0, jnp.float32)
        for j in range(_CHUNK // 16):
            ones_v[pl.ds(j * 16, 16)] = one
        zero = jnp.zeros((16,), jnp.float32)

        def zb_body(i, carry):
            zb[pl.ds(i * 16, 16)] = zero
            return carry

        lax.fori_loop(0, 2000 // 16, zb_body, 0)

        @pl.when(sid == 0)
        def _():
            for q in range(n // 2000):
                pltpu.sync_copy(zb, acc.at[pl.ds(q * 2000, 2000)])

        plsc.subcore_barrier()

        def blk_body(b, carry):
            pltpu.sync_copy(col_hbm.at[tid, b], col_v)

            def body(k, c2):
                pltpu.sync_copy(ones_v, acc.at[col_v.at[k]], add=True)
                return c2

            lax.fori_loop(0, bchunk, body, 0)
            return carry

        lax.fori_loop(0, nblk, blk_body, 0)
        plsc.subcore_barrier()

        @pl.when(sid == 0)
        def _():
            for q in range(n // 2000):
                pltpu.sync_copy(acc.at[pl.ds(q * 2000, 2000)], zb)
                pltpu.sync_copy(zb, out_hbm.at[pl.ds(cid * n + q * 2000, 2000)])

    return deg_kernel


@functools.lru_cache(maxsize=None)
def _make_scatter(nchunks, n):
    cpt = nchunks // (_NC * _NS)
    slabs = n // _CHUNK
    spt_lo = slabs // _NS
    extra = slabs % _NS
    mesh = plsc.VectorSubcoreMesh(core_axis_name="c", subcore_axis_name="s")

    nbuf = 4
    nblk = 5
    bchunk = cpt // nblk

    @functools.partial(
        pl.kernel,
        out_type=jax.ShapeDtypeStruct((_NC, n, _F), jnp.float32),
        mesh=mesh,
        scratch_types=(
            [pltpu.VMEM((bchunk, _CHUNK), jnp.int32)]
            + [pltpu.VMEM((_CHUNK,), jnp.int32)] * (2 * nbuf)
            + [pltpu.VMEM((_CHUNK, _F), jnp.float32)] * nbuf
            + [pltpu.VMEM_SHARED((n, _F), jnp.float32)]
            + [pltpu.SemaphoreType.DMA] * (2 * nbuf)
        ),
    )
    def scat_kernel(m_hbm, rc_hbm, out_hbm, rc_v, *rest):
        rbs = rest[0:2 * nbuf:2]
        cbs = rest[1:2 * nbuf:2]
        gbs = rest[2 * nbuf:3 * nbuf]
        acc = rest[3 * nbuf]
        sgs = rest[3 * nbuf + 1:3 * nbuf + 1 + nbuf]
        sss = rest[3 * nbuf + 1 + nbuf:3 * nbuf + 1 + 2 * nbuf]
        cid = lax.axis_index("c")
        sid = lax.axis_index("s")
        tid = cid * _NS + sid
        nslab = spt_lo + (sid < extra).astype(jnp.int32)

        def refill(k):
            @pl.when(lax.rem(k, bchunk) == 0)
            def _():
                pltpu.sync_copy(rc_hbm.at[tid, lax.div(k, bchunk)], rc_v)

        def unpack(k, rb, cb):
            kk = lax.rem(k, bchunk)
            for j in range(_CHUNK // 16):
                p = rc_v[kk, pl.ds(j * 16, 16)]
                rb[pl.ds(j * 16, 16)] = lax.shift_right_logical(p, 16)
                cb[pl.ds(j * 16, 16)] = lax.bitwise_and(p, 0xFFFF)

        def gath(b, sem):
            pltpu.async_copy(m_hbm.at[rbs[b]], gbs[b], sem)

        def gath_wait(b, sem):
            pltpu.make_async_copy(m_hbm.at[rbs[b]], gbs[b], sem).wait()

        def scat(b, sem):
            pltpu.async_copy(gbs[b], acc.at[cbs[b]], sem, add=True)

        def scat_wait(b, sem):
            pltpu.make_async_copy(gbs[b], acc.at[cbs[b]], sem).wait()

        pltpu.sync_copy(rc_hbm.at[tid, 0], rc_v)
        for b in range(nbuf - 1):
            unpack(b, rbs[b], cbs[b])
            gath(b, sgs[b])

        zero = jnp.zeros((16,), jnp.float32)
        groups = _F // 16

        def zb_body(i, carry):
            gbs[nbuf - 1][i // groups, pl.ds((i % groups) * 16, 16)] = zero
            return carry

        lax.fori_loop(0, _CHUNK * groups, zb_body, 0)

        def zslab_body(q, carry):
            slab = sid + q * _NS
            pltpu.sync_copy(gbs[nbuf - 1], acc.at[pl.ds(slab * _CHUNK, _CHUNK)])
            return carry

        lax.fori_loop(0, nslab, zslab_body, 0)
        unpack(nbuf - 1, rbs[nbuf - 1], cbs[nbuf - 1])
        gath(nbuf - 1, sgs[nbuf - 1])
        plsc.subcore_barrier()

        niters = (cpt + nbuf - 1) // nbuf

        def body(j, carry):
            base = nbuf * j
            for b in range(nbuf):
                k = base + b

                @pl.when(k < cpt)
                def _(b=b, k=k):
                    gath_wait(b, sgs[b])
                    scat(b, sss[b])

            for b in range(nbuf):
                k = base + b

                @pl.when(k + nbuf < cpt)
                def _(b=b, k=k):
                    scat_wait(b, sss[b])
                    refill(k + nbuf)
                    unpack(k + nbuf, rbs[b], cbs[b])
                    gath(b, sgs[b])

            return carry

        lax.fori_loop(0, niters, body, 0)
        for b in range(nbuf):
            scat_wait(b, sss[b])
        plsc.subcore_barrier()

        def ex_body(q, carry):
            slab = sid + q * _NS
            pltpu.sync_copy(acc.at[pl.ds(slab * _CHUNK, _CHUNK)],
                            out_hbm.at[cid, pl.ds(slab * _CHUNK, _CHUNK)])
            return carry

        lax.fori_loop(0, nslab, ex_body, 0)

    return scat_kernel


def _tc_pre(degp, x, W1):
    n = x.shape[0]

    def body(degp_ref, x_ref, w_ref, dis_ref, m_ref):
        deg = degp_ref[0] + degp_ref[1] + 1.0
        dis = lax.rsqrt(deg)
        dis_ref[...] = dis
        m_ref[...] = dis * _dot(x_ref[...], w_ref[...])

    return pl.pallas_call(
        body,
        out_shape=(jax.ShapeDtypeStruct((n, 1), jnp.float32),
                   jax.ShapeDtypeStruct((n, _F), jnp.float32)),
    )(degp, x, W1)


def _tc_mid(p, m, dis, b, Wn):
    n = m.shape[0]

    def body(p_ref, m_ref, dis_ref, b_ref, w_ref, out_ref):
        dis = dis_ref[...]
        pre = dis * (p_ref[0] + p_ref[1] + m_ref[...]) + b_ref[...]
        h = _leaky(pre)
        out_ref[...] = dis * _dot(h, w_ref[...])

    return pl.pallas_call(
        body,
        out_shape=jax.ShapeDtypeStruct((n, _F), jnp.float32),
    )(p, m, dis, b, Wn)


def _tc_final(p, m, dis, b3, batch, gfeat, Wg, bg, Wf, bf, Wm1, bm1, Wm2,
              bm2, alpha):
    n = m.shape[0]
    g = gfeat.shape[0]

    def _ln(v):
        mu = jnp.mean(v, axis=-1, keepdims=True)
        var = jnp.mean((v - mu) ** 2, axis=-1, keepdims=True)
        return (v - mu) * lax.rsqrt(var + 1e-5)

    def body(p_ref, m_ref, dis_ref, b_ref, batch_ref, gf_ref, wg_ref, bg_ref,
             wf_ref, bf_ref, wm1_ref, bm1_ref, wm2_ref, bm2_ref, a_ref,
             out_ref):
        dis = dis_ref[...]
        pre = dis * (p_ref[0] + p_ref[1] + m_ref[...]) + b_ref[...]
        h = _leaky(pre)
        gid = lax.broadcasted_iota(jnp.int32, (g, n), 0)
        onehot = (gid == batch_ref[...]).astype(jnp.float32)
        sums = _dot(onehot, h)
        cnt = jnp.sum(onehot, axis=1, keepdims=True)
        pooled = sums / jnp.maximum(cnt, 1.0)
        alpha = 1.0 / (1.0 + jnp.exp(-a_ref[0, 0]))
        gnn = _ln(_dot(pooled, wg_ref[...]) + bg_ref[...])
        gfe = _ln(_dot(gf_ref[...], wf_ref[...]) + bf_ref[...])
        fused = jnp.concatenate([gnn * alpha, gfe * (1.0 - alpha)], axis=1)
        o = _leaky(fused)
        o = _leaky(_dot(o, wm1_ref[...]) + bm1_ref[...])
        out_ref[...] = _dot(o, wm2_ref[...]) + bm2_ref[...]

    out_f = Wm2.shape[1]
    return pl.pallas_call(
        body,
        out_shape=jax.ShapeDtypeStruct((g, out_f), jnp.float32),
    )(p, m, dis, b3, batch, gfeat, Wg, bg, Wf, bf, Wm1, bm1, Wm2, bm2, alpha)


def kernel(x, edge_index, batch, graph_feature, W1, b1, W2, b2, W3, b3,
           Wg, bg, Wf, bf, Wm1, bm1, Wm2, bm2, alpha_param):
    n = x.shape[0]
    e = edge_index.shape[1]
    nw = _NC * _NS
    cpt = e // (nw * _CHUNK)
    rc3 = (lax.shift_left(edge_index[0], 16) |
           edge_index[1]).reshape(nw, 5, cpt // 5, _CHUNK)
    col4 = edge_index[1].reshape(nw, 5, cpt // 5, _CHUNK)

    degp = _make_degree(e // _CHUNK, n)(col4)
    degp = degp.reshape(_NC, n, 1)
    dis, m1 = _tc_pre(degp, x, W1)

    scat = _make_scatter(e // _CHUNK, n)
    p1 = scat(m1, rc3)
    m2 = _tc_mid(p1, m1, dis, b1.reshape(1, _F), W2)
    p2 = scat(m2, rc3)
    m3 = _tc_mid(p2, m2, dis, b2.reshape(1, _F), W3)
    p3 = scat(m3, rc3)

    return _tc_final(p3, m3, dis, b3.reshape(1, _F), batch.reshape(1, n),
                     graph_feature, Wg, bg.reshape(1, _F), Wf,
                     bf.reshape(1, _F), Wm1, bm1.reshape(1, _F), Wm2,
                     bm2.reshape(1, -1), alpha_param.reshape(1, 1))
